# baseline (speedup 1.0000x reference)
import functools

import jax
import jax.numpy as jnp
from jax import lax
from jax.experimental import pallas as pl
from jax.experimental.pallas import tpu as pltpu
from jax.experimental.pallas import tpu_sc as plsc

B = 16384
L = 50
D = 64
LANES = 16
NC, NS = 2, 16
NW = NC * NS
BPW = B // NW
CH = 1
RPC = CH * L
NCH = BPW // CH
NBUF = 8


def _bag_body(idx_hbm, w_hbm, emb_hbm, out_hbm, idx_v, w_v, rows_v, out_v, sems):
    wid = lax.axis_index("s") * NC + lax.axis_index("c")
    bag0 = wid * BPW

    pltpu.sync_copy(idx_hbm.at[pl.ds(wid * NCH, NCH)], idx_v)
    pltpu.sync_copy(w_hbm.at[pl.ds(bag0, BPW)], w_v)

    def start_gather(c, s):
        pltpu.make_async_copy(
            emb_hbm.at[idx_v.at[c]], rows_v.at[s], sems.at[s]
        ).start()

    for s in range(NBUF):
        start_gather(s, s)

    def outer(i, carry):
        cbase = i * NBUF
        for s in range(NBUF):
            c = cbase + s
            pltpu.make_async_copy(
                emb_hbm.at[idx_v.at[c]], rows_v.at[s], sems.at[s]
            ).wait()
            for k in range(CH):
                bag = c * CH + k
                wv = [w_v[bag, pl.ds(LANES * t, LANES)] for t in range(4)]
                acc = [jnp.zeros((LANES,), jnp.float32) for _ in range(4)]
                for l in range(L):
                    lane = jnp.full((LANES,), l % LANES, dtype=jnp.int32)
                    wb = jnp.take_along_axis(
                        wv[l // LANES], lane, axis=0, mode="promise_in_bounds"
                    )
                    r = k * L + l
                    for j in range(4):
                        acc[j] = acc[j] + wb * rows_v[s, r, pl.ds(LANES * j, LANES)]
                for j in range(4):
                    out_v[bag, pl.ds(LANES * j, LANES)] = acc[j]
            nxt = c + NBUF

            @pl.when(nxt < NCH)
            def _():
                start_gather(nxt, s)
        return carry

    lax.fori_loop(0, NCH // NBUF, outer, 0)
    pltpu.sync_copy(out_v, out_hbm.at[pl.ds(bag0, BPW)])


_bag_kernel = functools.partial(
    pl.kernel,
    out_type=jax.ShapeDtypeStruct((B, D), jnp.float32),
    mesh=plsc.VectorSubcoreMesh(core_axis_name="c", subcore_axis_name="s"),
    compiler_params=pltpu.CompilerParams(use_tc_tiling_on_sc=False),
    scratch_types=[
        pltpu.VMEM((NCH, RPC), jnp.int32),
        pltpu.VMEM((BPW, D), jnp.float32),
        pltpu.VMEM((NBUF, RPC, D), jnp.float32),
        pltpu.VMEM((BPW, D), jnp.float32),
        pltpu.SemaphoreType.DMA((NBUF,)),
    ],
)(_bag_body)


TBLK = 8192
_NB_IN = (1000000 + TBLK - 1) // TBLK
_NPB = (1000000 + 2 * TBLK - 1) // (2 * TBLK)
NPAIR = _NPB * TBLK
VLIN = 2 * NPAIR


def _transpose_body(a_ref, b_ref, y_ref):
    y_ref[...] = jnp.concatenate([a_ref[...].T, b_ref[...].T], axis=1)


def _transpose_tc(emb_t):
    return pl.pallas_call(
        _transpose_body,
        grid=(_NPB,),
        in_specs=[
            pl.BlockSpec((D, TBLK), lambda i: (0, 2 * i)),
            pl.BlockSpec((D, TBLK), lambda i: (0, jnp.minimum(2 * i + 1, _NB_IN - 1))),
        ],
        out_specs=pl.BlockSpec((TBLK, 2 * D), lambda i: (i, 0)),
        out_shape=jax.ShapeDtypeStruct((NPAIR, 2 * D), jnp.float32),
    )(emb_t, emb_t)


def kernel(indices, weights, embeddings):
    v = indices.astype(jnp.int32)
    tsh = TBLK.bit_length() - 1
    q = v >> tsh
    p = ((q & ~1) << tsh) + ((v & (TBLK - 1)) << 1) + (q & 1)
    idx2 = p
    w_pad = jnp.pad(weights, ((0, 0), (0, D - L)))
    emb_lin = _transpose_tc(embeddings.T).reshape(VLIN, D)
    return _bag_kernel(idx2, w_pad, emb_lin)


# speedup vs baseline: 1.0303x; 1.0303x over previous
import functools

import jax
import jax.numpy as jnp
from jax import lax
from jax.experimental import pallas as pl
from jax.experimental.pallas import tpu as pltpu
from jax.experimental.pallas import tpu_sc as plsc

B = 16384
L = 50
D = 64
LANES = 16
NC, NS = 2, 16
NW = NC * NS
BPW = B // NW
CH = 1
RPC = CH * L
NCH = BPW // CH
NBUF = 8


def _bag_body(idx_hbm, w_hbm, emb_hbm, out_hbm, idx_v, w_v, rows_v, out_v, sems):
    wid = lax.axis_index("s") * NC + lax.axis_index("c")
    bag0 = wid * BPW

    pltpu.sync_copy(idx_hbm.at[pl.ds(wid * NCH, NCH)], idx_v)
    pltpu.sync_copy(w_hbm.at[pl.ds(bag0, BPW)], w_v)

    def start_gather(c, s):
        pltpu.make_async_copy(
            emb_hbm.at[idx_v.at[c]], rows_v.at[s], sems.at[s]
        ).start()

    for s in range(NBUF):
        start_gather(s, s)

    def outer(i, carry):
        cbase = i * NBUF
        for s in range(NBUF):
            c = cbase + s
            pltpu.make_async_copy(
                emb_hbm.at[idx_v.at[c]], rows_v.at[s], sems.at[s]
            ).wait()
            for k in range(CH):
                bag = c * CH + k
                wv = [w_v[bag, pl.ds(LANES * t, LANES)] for t in range(4)]
                acc = [jnp.zeros((LANES,), jnp.float32) for _ in range(4)]
                for l in range(L):
                    lane = jnp.full((LANES,), l % LANES, dtype=jnp.int32)
                    wb = jnp.take_along_axis(
                        wv[l // LANES], lane, axis=0, mode="promise_in_bounds"
                    )
                    r = k * L + l
                    for j in range(4):
                        acc[j] = acc[j] + wb * rows_v[s, r, pl.ds(LANES * j, LANES)]
                for j in range(4):
                    out_v[bag, pl.ds(LANES * j, LANES)] = acc[j]
            nxt = c + NBUF

            @pl.when(nxt < NCH)
            def _():
                start_gather(nxt, s)
        return carry

    lax.fori_loop(0, NCH // NBUF, outer, 0)
    pltpu.sync_copy(out_v, out_hbm.at[pl.ds(bag0, BPW)])


_bag_kernel = functools.partial(
    pl.kernel,
    out_type=jax.ShapeDtypeStruct((B, D), jnp.float32),
    mesh=plsc.VectorSubcoreMesh(core_axis_name="c", subcore_axis_name="s"),
    compiler_params=pltpu.CompilerParams(use_tc_tiling_on_sc=False),
    scratch_types=[
        pltpu.VMEM((NCH, RPC), jnp.int32),
        pltpu.VMEM((BPW, D), jnp.float32),
        pltpu.VMEM((NBUF, RPC, D), jnp.float32),
        pltpu.VMEM((BPW, D), jnp.float32),
        pltpu.SemaphoreType.DMA((NBUF,)),
    ],
)(_bag_body)


TBLK = 16384
_NB_IN = (1000000 + TBLK - 1) // TBLK
_NPB = (1000000 + 2 * TBLK - 1) // (2 * TBLK)
NPAIR = _NPB * TBLK
VLIN = 2 * NPAIR


def _transpose_body(a_ref, b_ref, y_ref):
    y_ref[...] = jnp.concatenate([a_ref[...].T, b_ref[...].T], axis=1)


def _transpose_tc(emb_t):
    return pl.pallas_call(
        _transpose_body,
        grid=(_NPB,),
        in_specs=[
            pl.BlockSpec((D, TBLK), lambda i: (0, 2 * i)),
            pl.BlockSpec((D, TBLK), lambda i: (0, jnp.minimum(2 * i + 1, _NB_IN - 1))),
        ],
        out_specs=pl.BlockSpec((TBLK, 2 * D), lambda i: (i, 0)),
        out_shape=jax.ShapeDtypeStruct((NPAIR, 2 * D), jnp.float32),
    )(emb_t, emb_t)


def kernel(indices, weights, embeddings):
    v = indices.astype(jnp.int32)
    tsh = TBLK.bit_length() - 1
    q = v >> tsh
    p = ((q & ~1) << tsh) + ((v & (TBLK - 1)) << 1) + (q & 1)
    idx2 = p
    w_pad = jnp.pad(weights, ((0, 0), (0, D - L)))
    emb_lin = _transpose_tc(embeddings.T).reshape(VLIN, D)
    return _bag_kernel(idx2, w_pad, emb_lin)
